# Initial kernel scaffold; baseline (speedup 1.0000x reference)
import jax
import jax.numpy as jnp
from jax.experimental import pallas as pl


def kernel(x, tok_table, pos_table):
    raise NotImplementedError("write your pallas kernel here")


# speedup vs baseline: 1.1540x; 1.1540x over previous
import functools

import jax
import jax.numpy as jnp
from jax import lax
from jax.experimental import pallas as pl
from jax.experimental.pallas import tpu as pltpu
from jax.experimental.pallas import tpu_sc as plsc

VOCAB_SIZE = 50257
EMBED = 768
BATCH = 4
SEQ = 2048
NTOK = BATCH * SEQ

NUM_CORES = 2
NUM_SUBCORES = 16
NUM_WORKERS = NUM_CORES * NUM_SUBCORES
LANES = 16

POS_BLK = SEQ // NUM_WORKERS
COLS = EMBED // LANES

_mesh = plsc.VectorSubcoreMesh(core_axis_name="c", subcore_axis_name="s")


@functools.partial(
    pl.kernel,
    mesh=_mesh,
    out_type=jax.ShapeDtypeStruct((NTOK, EMBED), jnp.float32),
    scratch_types=[
        pltpu.VMEM((POS_BLK,), jnp.int32),
        pltpu.VMEM((POS_BLK, EMBED), jnp.float32),
        pltpu.VMEM((POS_BLK, EMBED), jnp.float32),
        pltpu.SemaphoreType.DMA,
    ],
)
def _embed_sc(x_hbm, tok_hbm, pos_hbm, out_hbm, idx_v, tok_v, pos_v, sem):
    wid = lax.axis_index("s") * NUM_CORES + lax.axis_index("c")
    pbase = wid * POS_BLK

    pltpu.sync_copy(pos_hbm.at[pl.ds(pbase, POS_BLK)], pos_v)

    for b in range(BATCH):
        tbase = b * SEQ + pbase
        pltpu.sync_copy(x_hbm.at[pl.ds(tbase, POS_BLK)], idx_v)
        pltpu.async_copy(tok_hbm.at[idx_v], tok_v, sem).wait()

        def _row(r, carry):
            for c in range(COLS):
                sl = pl.ds(c * LANES, LANES)
                tok_v[r, sl] = tok_v[r, sl] + pos_v[r, sl]
            return carry

        lax.fori_loop(0, POS_BLK, _row, 0)
        pltpu.sync_copy(tok_v, out_hbm.at[pl.ds(tbase, POS_BLK)])


@jax.jit
def kernel(x, tok_table, pos_table):
    out = _embed_sc(x.reshape(-1), tok_table, pos_table)
    return out.reshape(BATCH, SEQ, EMBED)
